# Initial kernel scaffold; baseline (speedup 1.0000x reference)
import jax
import jax.numpy as jnp
from jax.experimental import pallas as pl


def kernel(x, edge_index, edge_values, W_lin, b_lin, W_agg, Wk, Wq, Wv, ln_attn_g, ln_attn_b, W_final, b_final, ln_gps_g, ln_gps_b):
    raise NotImplementedError("write your pallas kernel here")


# speedup vs baseline: 4.0243x; 4.0243x over previous
import functools

import jax
import jax.numpy as jnp
from jax import lax
from jax.experimental import pallas as pl
from jax.experimental.pallas import tpu as pltpu
from jax.experimental.pallas import tpu_sc as plsc

N = 10000
E = 320000
D = 128
L = 2
NS = 16
EPS = E // NS
C = 80
NCHUNK = EPS // C
RPS = 624
TAIL = N - NS * RPS
ZR = 208

@functools.cache
def _make_spmm():
    mesh = plsc.VectorSubcoreMesh(core_axis_name="c", subcore_axis_name="s",
                                  num_cores=2, num_subcores=NS)
    return pl.kernel(
        _spmm_body,
        out_type=(jax.ShapeDtypeStruct((N, D), jnp.float32),
                  jax.ShapeDtypeStruct((N, D), jnp.float32)),
        mesh=mesh,
        scratch_types=[
            pltpu.VMEM_SHARED((N, D), jnp.float32),
            pltpu.VMEM((C,), jnp.int32),
            pltpu.VMEM((C,), jnp.int32),
            pltpu.VMEM((C,), jnp.float32),
            pltpu.VMEM((C, D), jnp.float32),
            pltpu.VMEM((ZR, D), jnp.float32),
            pltpu.SemaphoreType.DMA,
        ],
    )


def _spmm_body(dst_hbm, src_hbm, val_hbm, xa_hbm, xb_hbm, ya_hbm, yb_hbm,
               acc_sh, src_v, dst_v, val_v, rows_v, zbuf, sem):
    c = lax.axis_index("c")
    s = lax.axis_index("s")
    zeros16 = jnp.zeros((16,), jnp.float32)

    @pl.loop(0, ZR)
    def _zero(r):
        for j in range(D // 16):
            zbuf[r, pl.ds(j * 16, 16)] = zeros16

    row0 = s * RPS
    for k in range(RPS // ZR):
        pltpu.sync_copy(zbuf, acc_sh.at[pl.ds(row0 + k * ZR, ZR)])

    @pl.when(s == 0)
    def _():
        pltpu.sync_copy(zbuf.at[pl.ds(0, TAIL)], acc_sh.at[pl.ds(NS * RPS, TAIL)])

    plsc.subcore_barrier()

    ebase = s * EPS

    @pl.loop(0, NCHUNK)
    def _chunk(i):
        base = ebase + i * C
        pltpu.sync_copy(src_hbm.at[pl.ds(base, C)], src_v)
        pltpu.sync_copy(dst_hbm.at[pl.ds(base, C)], dst_v)
        pltpu.sync_copy(val_hbm.at[pl.ds(base, C)], val_v)

        @pl.when(c == 0)
        def _():
            pltpu.async_copy(xa_hbm.at[src_v], rows_v, sem).wait()

        @pl.when(c == 1)
        def _():
            pltpu.async_copy(xb_hbm.at[src_v], rows_v, sem).wait()

        @pl.loop(0, C // 16)
        def _scale(g):
            vals16 = val_v[pl.ds(g * 16, 16)]
            for j in range(16):
                v = vals16[j]
                e = g * 16 + j
                for k in range(D // 16):
                    sl = pl.ds(k * 16, 16)
                    rows_v[e, sl] = rows_v[e, sl] * v

        pltpu.sync_copy(rows_v, acc_sh.at[dst_v], add=True)

    plsc.subcore_barrier()

    @pl.when(c == 0)
    def _():
        pltpu.sync_copy(acc_sh.at[pl.ds(row0, RPS)], ya_hbm.at[pl.ds(row0, RPS)])

        @pl.when(s == 0)
        def _():
            pltpu.sync_copy(acc_sh.at[pl.ds(NS * RPS, TAIL)],
                            ya_hbm.at[pl.ds(NS * RPS, TAIL)])

    @pl.when(c == 1)
    def _():
        pltpu.sync_copy(acc_sh.at[pl.ds(row0, RPS)], yb_hbm.at[pl.ds(row0, RPS)])

        @pl.when(s == 0)
        def _():
            pltpu.sync_copy(acc_sh.at[pl.ds(NS * RPS, TAIL)],
                            yb_hbm.at[pl.ds(NS * RPS, TAIL)])


NB = 1000


def _epilogue_body(x_ref, ya_ref, yb_ref, y2a_ref, y2b_ref,
                   wl_ref, wa_ref, wk_ref, wq_ref, wv_ref, wf_ref,
                   bl_ref, bf_ref, ag_ref, ab_ref, gg_ref, gb_ref, o_ref):
    wl = wl_ref[...]
    wa = wa_ref[...]
    wk = wk_ref[...]
    wq = wq_ref[...]
    wv = wv_ref[...]
    wf = wf_ref[...]
    bl = bl_ref[...]
    bf = bf_ref[...]
    ag = ag_ref[...]
    ab = ab_ref[...]
    gg = gg_ref[...]
    gb = gb_ref[...]

    def dot_t(a, w):
        return lax.dot_general(a, w, (((1,), (1,)), ((), ())),
                               preferred_element_type=jnp.float32)

    def ln(t, g, b):
        mu = jnp.mean(t, axis=-1, keepdims=True)
        var = jnp.mean((t - mu) ** 2, axis=-1, keepdims=True)
        return (t - mu) / jnp.sqrt(var + 1e-5) * g + b

    def silu(t):
        return t * jax.nn.sigmoid(t)

    y1_refs = (ya_ref, yb_ref)
    y2_refs = (y2a_ref, y2b_ref)
    for l in range(L):
        xl = x_ref[l]
        y1l = y1_refs[l][...]
        y2l = y2_refs[l][...]
        proj = dot_t(xl, wl) + bl
        sage = silu(proj + dot_t(y1l, wa))
        xr1 = 0.5 * y1l
        xr2 = 0.25 * y2l

        def wmean(t):
            return jnp.mean(dot_t(t, wk) * dot_t(t, wq), axis=-1, keepdims=True)

        w = wmean(xl) + wmean(xr1) + wmean(xr2)
        xo = xl + xr1 + xr2
        attn = ln(dot_t(xo, wv) * w, ag, ab)
        h = silu(dot_t(sage + attn, wf) + bf)
        o_ref[l] = ln(h, gg, gb)


def _epilogue(x, ya, yb, y2a, y2b, W_lin, W_agg, Wk, Wq, Wv, W_final,
              b_lin, b_final, ln_attn_g, ln_attn_b, ln_gps_g, ln_gps_b):
    full = lambda shape: pl.BlockSpec(shape, lambda i: tuple(0 for _ in shape))
    node = pl.BlockSpec((NB, D), lambda i: (i, 0))
    return pl.pallas_call(
        _epilogue_body,
        grid=(N // NB,),
        in_specs=[
            pl.BlockSpec((L, NB, D), lambda i: (0, i, 0)),
            node, node, node, node,
            full((D, D)), full((D, D)), full(Wk.shape), full(Wq.shape),
            full((D, D)), full((D, D)),
            full((1, D)), full((1, D)), full((1, D)), full((1, D)),
            full((1, D)), full((1, D)),
        ],
        out_specs=pl.BlockSpec((L, NB, D), lambda i: (0, i, 0)),
        out_shape=jax.ShapeDtypeStruct((L, N, D), jnp.float32),
    )(x, ya, yb, y2a, y2b, W_lin, W_agg, Wk, Wq, Wv, W_final,
      b_lin.reshape(1, D), b_final.reshape(1, D),
      ln_attn_g.reshape(1, D), ln_attn_b.reshape(1, D),
      ln_gps_g.reshape(1, D), ln_gps_b.reshape(1, D))


def kernel(x, edge_index, edge_values, W_lin, b_lin, W_agg, Wk, Wq, Wv,
           ln_attn_g, ln_attn_b, W_final, b_final, ln_gps_g, ln_gps_b):
    dst = edge_index[0]
    src = edge_index[1]
    spmm = _make_spmm()
    ya, yb = spmm(dst, src, edge_values, x[0], x[1])
    y2a, y2b = spmm(dst, src, edge_values, ya, yb)
    return _epilogue(x, ya, yb, y2a, y2b, W_lin, W_agg, Wk, Wq, Wv, W_final,
                     b_lin, b_final, ln_attn_g, ln_attn_b, ln_gps_g, ln_gps_b)


# speedup vs baseline: 4.2782x; 1.0631x over previous
import functools

import jax
import jax.numpy as jnp
from jax import lax
from jax.experimental import pallas as pl
from jax.experimental.pallas import tpu as pltpu
from jax.experimental.pallas import tpu_sc as plsc

N = 10000
E = 320000
D = 128
L = 2
NS = 16
EPS = E // NS
C = 80
NCHUNK = EPS // C
RPS = 624
TAIL = N - NS * RPS
ZR = 16
NBUF = 3
IB = 8

@functools.cache
def _make_spmm():
    mesh = plsc.VectorSubcoreMesh(core_axis_name="c", subcore_axis_name="s",
                                  num_cores=2, num_subcores=NS)
    return pl.kernel(
        _spmm_body,
        out_type=(jax.ShapeDtypeStruct((N, D), jnp.float32),
                  jax.ShapeDtypeStruct((N, D), jnp.float32)),
        mesh=mesh,
        scratch_types=[
            pltpu.VMEM_SHARED((N, D), jnp.float32),
            pltpu.VMEM((IB, C), jnp.int32),
            pltpu.VMEM((IB, C), jnp.int32),
            pltpu.VMEM((IB, C), jnp.float32),
            pltpu.VMEM((NBUF, C, D), jnp.float32),
            pltpu.VMEM((ZR, D), jnp.float32),
            pltpu.SemaphoreType.DMA((NBUF,)),
            pltpu.SemaphoreType.DMA((NBUF,)),
            pltpu.SemaphoreType.DMA((IB,)),
            pltpu.SemaphoreType.DMA,
        ],
    )


def _spmm_body(src3_hbm, dst3_hbm, val3_hbm, xa_hbm, xb_hbm, ya_hbm, yb_hbm,
               acc_sh, src_v, dst_v, val_v, bufs, zbuf, gsem, ssem, isem, zsem):
    c = lax.axis_index("c")
    s = lax.axis_index("s")
    zeros16 = jnp.zeros((16,), jnp.float32)

    @pl.loop(0, ZR)
    def _zero(r):
        for j in range(D // 16):
            zbuf[r, pl.ds(j * 16, 16)] = zeros16

    row0 = s * RPS
    for k in range(RPS // ZR):
        pltpu.make_async_copy(zbuf, acc_sh.at[pl.ds(row0 + k * ZR, ZR)],
                              zsem).start()

    @pl.when(s == 0)
    def _():
        pltpu.make_async_copy(zbuf, acc_sh.at[pl.ds(NS * RPS, TAIL)],
                              zsem).start()

    def idx_start(i):
        k = lax.rem(i, IB)
        pltpu.async_copy(src3_hbm.at[s, i], src_v.at[k], isem.at[k])
        pltpu.async_copy(dst3_hbm.at[s, i], dst_v.at[k], isem.at[k])
        pltpu.async_copy(val3_hbm.at[s, i], val_v.at[k], isem.at[k])

    def idx_wait(i):
        k = lax.rem(i, IB)
        pltpu.make_async_copy(src3_hbm.at[s, i], src_v.at[k], isem.at[k]).wait()
        pltpu.make_async_copy(dst3_hbm.at[s, i], dst_v.at[k], isem.at[k]).wait()
        pltpu.make_async_copy(val3_hbm.at[s, i], val_v.at[k], isem.at[k]).wait()

    def g_start(i):
        k = lax.rem(i, IB)
        b = lax.rem(i, NBUF)

        @pl.when(c == 0)
        def _():
            pltpu.async_copy(xa_hbm.at[src_v.at[k]], bufs.at[b], gsem.at[b])

        @pl.when(c == 1)
        def _():
            pltpu.async_copy(xb_hbm.at[src_v.at[k]], bufs.at[b], gsem.at[b])

    def g_wait(i):
        k = lax.rem(i, IB)
        b = lax.rem(i, NBUF)

        @pl.when(c == 0)
        def _():
            pltpu.make_async_copy(xa_hbm.at[src_v.at[k]], bufs.at[b],
                                  gsem.at[b]).wait()

        @pl.when(c == 1)
        def _():
            pltpu.make_async_copy(xb_hbm.at[src_v.at[k]], bufs.at[b],
                                  gsem.at[b]).wait()

    def s_start(i):
        k = lax.rem(i, IB)
        b = lax.rem(i, NBUF)
        pltpu.make_async_copy(bufs.at[b], acc_sh.at[dst_v.at[k]],
                              ssem.at[b]).start(add=True)

    def s_wait(i):
        k = lax.rem(i, IB)
        b = lax.rem(i, NBUF)
        pltpu.make_async_copy(bufs.at[b], acc_sh.at[dst_v.at[k]],
                              ssem.at[b]).wait()

    def scale_chunk(i):
        k = lax.rem(i, IB)
        b = lax.rem(i, NBUF)

        @pl.loop(0, C // 16)
        def _scale(g):
            vals16 = val_v[k, pl.ds(g * 16, 16)]
            for j in range(16):
                v = vals16[j]
                e = g * 16 + j
                for q in range(D // 16):
                    sl = pl.ds(q * 16, 16)
                    bufs[b, e, sl] = bufs[b, e, sl] * v

    for i in range(4):
        idx_start(i)
    idx_wait(0)
    idx_wait(1)
    g_start(0)
    g_start(1)

    for k in range(RPS // ZR):
        pltpu.make_async_copy(zbuf, acc_sh.at[pl.ds(row0 + k * ZR, ZR)],
                              zsem).wait()

    @pl.when(s == 0)
    def _():
        pltpu.make_async_copy(zbuf, acc_sh.at[pl.ds(NS * RPS, TAIL)],
                              zsem).wait()

    plsc.subcore_barrier()

    @pl.loop(0, NCHUNK)
    def _step(i):
        g_wait(i)
        scale_chunk(i)
        s_start(i)

        @pl.when(i >= 1)
        def _():
            s_wait(i - 1)

        @pl.when(i + 4 < NCHUNK)
        def _():
            idx_start(i + 4)

        @pl.when(i + 2 < NCHUNK)
        def _():
            idx_wait(i + 2)
            g_start(i + 2)

    s_wait(NCHUNK - 1)
    plsc.subcore_barrier()

    @pl.when(c == 0)
    def _():
        pltpu.sync_copy(acc_sh.at[pl.ds(row0, RPS)], ya_hbm.at[pl.ds(row0, RPS)])

        @pl.when(s == 0)
        def _():
            pltpu.sync_copy(acc_sh.at[pl.ds(NS * RPS, TAIL)],
                            ya_hbm.at[pl.ds(NS * RPS, TAIL)])

    @pl.when(c == 1)
    def _():
        pltpu.sync_copy(acc_sh.at[pl.ds(row0, RPS)], yb_hbm.at[pl.ds(row0, RPS)])

        @pl.when(s == 0)
        def _():
            pltpu.sync_copy(acc_sh.at[pl.ds(NS * RPS, TAIL)],
                            yb_hbm.at[pl.ds(NS * RPS, TAIL)])


NB = 1000


def _epilogue_body(x_ref, ya_ref, yb_ref, y2a_ref, y2b_ref,
                   wl_ref, wa_ref, wk_ref, wq_ref, wv_ref, wf_ref,
                   bl_ref, bf_ref, ag_ref, ab_ref, gg_ref, gb_ref, o_ref):
    wl = wl_ref[...]
    wa = wa_ref[...]
    wk = wk_ref[...]
    wq = wq_ref[...]
    wv = wv_ref[...]
    wf = wf_ref[...]
    bl = bl_ref[...]
    bf = bf_ref[...]
    ag = ag_ref[...]
    ab = ab_ref[...]
    gg = gg_ref[...]
    gb = gb_ref[...]

    def dot_t(a, w):
        return lax.dot_general(a, w, (((1,), (1,)), ((), ())),
                               preferred_element_type=jnp.float32)

    def ln(t, g, b):
        mu = jnp.mean(t, axis=-1, keepdims=True)
        var = jnp.mean((t - mu) ** 2, axis=-1, keepdims=True)
        return (t - mu) / jnp.sqrt(var + 1e-5) * g + b

    def silu(t):
        return t * jax.nn.sigmoid(t)

    y1_refs = (ya_ref, yb_ref)
    y2_refs = (y2a_ref, y2b_ref)
    for l in range(L):
        xl = x_ref[l]
        y1l = y1_refs[l][...]
        y2l = y2_refs[l][...]
        proj = dot_t(xl, wl) + bl
        sage = silu(proj + dot_t(y1l, wa))
        xr1 = 0.5 * y1l
        xr2 = 0.25 * y2l

        def wmean(t):
            return jnp.mean(dot_t(t, wk) * dot_t(t, wq), axis=-1, keepdims=True)

        w = wmean(xl) + wmean(xr1) + wmean(xr2)
        xo = xl + xr1 + xr2
        attn = ln(dot_t(xo, wv) * w, ag, ab)
        h = silu(dot_t(sage + attn, wf) + bf)
        o_ref[l] = ln(h, gg, gb)


def _epilogue(x, ya, yb, y2a, y2b, W_lin, W_agg, Wk, Wq, Wv, W_final,
              b_lin, b_final, ln_attn_g, ln_attn_b, ln_gps_g, ln_gps_b):
    full = lambda shape: pl.BlockSpec(shape, lambda i: tuple(0 for _ in shape))
    node = pl.BlockSpec((NB, D), lambda i: (i, 0))
    return pl.pallas_call(
        _epilogue_body,
        grid=(N // NB,),
        in_specs=[
            pl.BlockSpec((L, NB, D), lambda i: (0, i, 0)),
            node, node, node, node,
            full((D, D)), full((D, D)), full(Wk.shape), full(Wq.shape),
            full((D, D)), full((D, D)),
            full((1, D)), full((1, D)), full((1, D)), full((1, D)),
            full((1, D)), full((1, D)),
        ],
        out_specs=pl.BlockSpec((L, NB, D), lambda i: (0, i, 0)),
        out_shape=jax.ShapeDtypeStruct((L, N, D), jnp.float32),
    )(x, ya, yb, y2a, y2b, W_lin, W_agg, Wk, Wq, Wv, W_final,
      b_lin.reshape(1, D), b_final.reshape(1, D),
      ln_attn_g.reshape(1, D), ln_attn_b.reshape(1, D),
      ln_gps_g.reshape(1, D), ln_gps_b.reshape(1, D))


def kernel(x, edge_index, edge_values, W_lin, b_lin, W_agg, Wk, Wq, Wv,
           ln_attn_g, ln_attn_b, W_final, b_final, ln_gps_g, ln_gps_b):
    src3 = edge_index[1].reshape(NS, NCHUNK, C)
    dst3 = edge_index[0].reshape(NS, NCHUNK, C)
    val3 = edge_values.reshape(NS, NCHUNK, C)
    spmm = _make_spmm()
    ya, yb = spmm(src3, dst3, val3, x[0], x[1])
    y2a, y2b = spmm(src3, dst3, val3, ya, yb)
    return _epilogue(x, ya, yb, y2a, y2b, W_lin, W_agg, Wk, Wq, Wv, W_final,
                     b_lin, b_final, ln_attn_g, ln_attn_b, ln_gps_g, ln_gps_b)


# speedup vs baseline: 5.4076x; 1.2640x over previous
import functools

import jax
import jax.numpy as jnp
from jax import lax
from jax.experimental import pallas as pl
from jax.experimental.pallas import tpu as pltpu
from jax.experimental.pallas import tpu_sc as plsc

N = 10000
E = 320000
D = 128
L = 2
NS = 16
C = 80
NCHUNK = 256
EPAD = NS * NCHUNK * C
EPS = NCHUNK * C
RPS = 624
TAIL = N - NS * RPS
ZR = 16
NBUF = 4
IB = 8

@functools.cache
def _make_spmm():
    mesh = plsc.VectorSubcoreMesh(core_axis_name="c", subcore_axis_name="s",
                                  num_cores=2, num_subcores=NS)
    return pl.kernel(
        _spmm_body,
        out_type=(jax.ShapeDtypeStruct((N, D), jnp.float32),
                  jax.ShapeDtypeStruct((N, D), jnp.float32)),
        mesh=mesh,
        scratch_types=[
            pltpu.VMEM_SHARED((N, D), jnp.float32),
            pltpu.VMEM((IB, C), jnp.int32),
            pltpu.VMEM((IB, C), jnp.int32),
            pltpu.VMEM((IB, C), jnp.float32),
            pltpu.VMEM((NBUF, C, D), jnp.float32),
            pltpu.VMEM((ZR, D), jnp.float32),
            pltpu.SemaphoreType.DMA((NBUF,)),
            pltpu.SemaphoreType.DMA((NBUF,)),
            pltpu.SemaphoreType.DMA((IB,)),
            pltpu.SemaphoreType.DMA,
        ],
    )


def _spmm_body(src3_hbm, dst3_hbm, val3_hbm, xa_hbm, xb_hbm, ya_hbm, yb_hbm,
               acc_sh, src_v, dst_v, val_v, bufs, zbuf, gsem, ssem, isem, zsem):
    c = lax.axis_index("c")
    s = lax.axis_index("s")
    zeros16 = jnp.zeros((16,), jnp.float32)

    @pl.loop(0, ZR)
    def _zero(r):
        for j in range(D // 16):
            zbuf[r, pl.ds(j * 16, 16)] = zeros16

    row0 = s * RPS
    for k in range(RPS // ZR):
        pltpu.make_async_copy(zbuf, acc_sh.at[pl.ds(row0 + k * ZR, ZR)],
                              zsem).start()

    @pl.when(s == 0)
    def _():
        pltpu.make_async_copy(zbuf, acc_sh.at[pl.ds(NS * RPS, TAIL)],
                              zsem).start()

    def idx_start(i, k):
        pltpu.async_copy(src3_hbm.at[s, i], src_v.at[k], isem.at[k])
        pltpu.async_copy(dst3_hbm.at[s, i], dst_v.at[k], isem.at[k])
        pltpu.async_copy(val3_hbm.at[s, i], val_v.at[k], isem.at[k])

    def idx_wait(i, k):
        pltpu.make_async_copy(src3_hbm.at[s, i], src_v.at[k], isem.at[k]).wait()
        pltpu.make_async_copy(dst3_hbm.at[s, i], dst_v.at[k], isem.at[k]).wait()
        pltpu.make_async_copy(val3_hbm.at[s, i], val_v.at[k], isem.at[k]).wait()

    def g_start(k, b):
        @pl.when(c == 0)
        def _():
            pltpu.async_copy(xa_hbm.at[src_v.at[k]], bufs.at[b], gsem.at[b])

        @pl.when(c == 1)
        def _():
            pltpu.async_copy(xb_hbm.at[src_v.at[k]], bufs.at[b], gsem.at[b])

    def g_wait(k, b):
        @pl.when(c == 0)
        def _():
            pltpu.make_async_copy(xa_hbm.at[src_v.at[k]], bufs.at[b],
                                  gsem.at[b]).wait()

        @pl.when(c == 1)
        def _():
            pltpu.make_async_copy(xb_hbm.at[src_v.at[k]], bufs.at[b],
                                  gsem.at[b]).wait()

    def s_start(k, b):
        pltpu.make_async_copy(bufs.at[b], acc_sh.at[dst_v.at[k]],
                              ssem.at[b]).start(add=True)

    def s_wait(k, b):
        pltpu.make_async_copy(bufs.at[b], acc_sh.at[dst_v.at[k]],
                              ssem.at[b]).wait()

    def scale_chunk(k, b):
        @pl.loop(0, C // 16)
        def _scale(g):
            vals16 = val_v[k, pl.ds(g * 16, 16)]
            for j in range(16):
                v = vals16[j]
                e = g * 16 + j
                sls = [pl.ds(q * 16, 16) for q in range(D // 16)]
                rows = [bufs[b, e, sl] for sl in sls]
                rows = [r * v for r in rows]
                for sl, r in zip(sls, rows):
                    bufs[b, e, sl] = r

    for i in range(4):
        idx_start(i, i)
    idx_wait(0, 0)
    idx_wait(1, 1)
    g_start(0, 0)
    g_start(1, 1)

    for k in range(RPS // ZR):
        pltpu.make_async_copy(zbuf, acc_sh.at[pl.ds(row0 + k * ZR, ZR)],
                              zsem).wait()

    @pl.when(s == 0)
    def _():
        pltpu.make_async_copy(zbuf, acc_sh.at[pl.ds(NS * RPS, TAIL)],
                              zsem).wait()

    plsc.subcore_barrier()

    @pl.loop(0, NCHUNK // IB)
    def _block(t):
        i0 = t * IB
        for db in range(IB):
            i = i0 + db
            b = db % NBUF
            g_wait(db, b)
            scale_chunk(db, b)
            s_start(db, b)
            if db == 0:
                @pl.when(t > 0)
                def _():
                    s_wait(IB - 1, NBUF - 1)
            else:
                s_wait(db - 1, (db - 1) % NBUF)

            @pl.when(i + 4 < NCHUNK)
            def _():
                idx_start(i + 4, (db + 4) % IB)

            @pl.when(i + 2 < NCHUNK)
            def _():
                idx_wait(i + 2, (db + 2) % IB)
                g_start((db + 2) % IB, (db + 2) % NBUF)

    s_wait(IB - 1, NBUF - 1)
    plsc.subcore_barrier()

    @pl.when(c == 0)
    def _():
        pltpu.sync_copy(acc_sh.at[pl.ds(row0, RPS)], ya_hbm.at[pl.ds(row0, RPS)])

        @pl.when(s == 0)
        def _():
            pltpu.sync_copy(acc_sh.at[pl.ds(NS * RPS, TAIL)],
                            ya_hbm.at[pl.ds(NS * RPS, TAIL)])

    @pl.when(c == 1)
    def _():
        pltpu.sync_copy(acc_sh.at[pl.ds(row0, RPS)], yb_hbm.at[pl.ds(row0, RPS)])

        @pl.when(s == 0)
        def _():
            pltpu.sync_copy(acc_sh.at[pl.ds(NS * RPS, TAIL)],
                            yb_hbm.at[pl.ds(NS * RPS, TAIL)])


NB = 1000


def _epilogue_body(x_ref, ya_ref, yb_ref, y2a_ref, y2b_ref,
                   wl_ref, wa_ref, wk_ref, wq_ref, wv_ref, wf_ref,
                   bl_ref, bf_ref, ag_ref, ab_ref, gg_ref, gb_ref, o_ref):
    wl = wl_ref[...]
    wa = wa_ref[...]
    wk = wk_ref[...]
    wq = wq_ref[...]
    wv = wv_ref[...]
    wf = wf_ref[...]
    bl = bl_ref[...]
    bf = bf_ref[...]
    ag = ag_ref[...]
    ab = ab_ref[...]
    gg = gg_ref[...]
    gb = gb_ref[...]

    def dot_t(a, w):
        return lax.dot_general(a, w, (((1,), (1,)), ((), ())),
                               preferred_element_type=jnp.float32)

    def ln(t, g, b):
        mu = jnp.mean(t, axis=-1, keepdims=True)
        var = jnp.mean((t - mu) ** 2, axis=-1, keepdims=True)
        return (t - mu) / jnp.sqrt(var + 1e-5) * g + b

    def silu(t):
        return t * jax.nn.sigmoid(t)

    y1_refs = (ya_ref, yb_ref)
    y2_refs = (y2a_ref, y2b_ref)
    for l in range(L):
        xl = x_ref[l]
        y1l = y1_refs[l][...]
        y2l = y2_refs[l][...]
        proj = dot_t(xl, wl) + bl
        sage = silu(proj + dot_t(y1l, wa))
        xr1 = 0.5 * y1l
        xr2 = 0.25 * y2l

        def wmean(t):
            return jnp.mean(dot_t(t, wk) * dot_t(t, wq), axis=-1, keepdims=True)

        w = wmean(xl) + wmean(xr1) + wmean(xr2)
        xo = xl + xr1 + xr2
        attn = ln(dot_t(xo, wv) * w, ag, ab)
        h = silu(dot_t(sage + attn, wf) + bf)
        o_ref[l] = ln(h, gg, gb)


def _epilogue(x, ya, yb, y2a, y2b, W_lin, W_agg, Wk, Wq, Wv, W_final,
              b_lin, b_final, ln_attn_g, ln_attn_b, ln_gps_g, ln_gps_b):
    full = lambda shape: pl.BlockSpec(shape, lambda i: tuple(0 for _ in shape))
    node = pl.BlockSpec((NB, D), lambda i: (i, 0))
    return pl.pallas_call(
        _epilogue_body,
        grid=(N // NB,),
        in_specs=[
            pl.BlockSpec((L, NB, D), lambda i: (0, i, 0)),
            node, node, node, node,
            full((D, D)), full((D, D)), full(Wk.shape), full(Wq.shape),
            full((D, D)), full((D, D)),
            full((1, D)), full((1, D)), full((1, D)), full((1, D)),
            full((1, D)), full((1, D)),
        ],
        out_specs=pl.BlockSpec((L, NB, D), lambda i: (0, i, 0)),
        out_shape=jax.ShapeDtypeStruct((L, N, D), jnp.float32),
    )(x, ya, yb, y2a, y2b, W_lin, W_agg, Wk, Wq, Wv, W_final,
      b_lin.reshape(1, D), b_final.reshape(1, D),
      ln_attn_g.reshape(1, D), ln_attn_b.reshape(1, D),
      ln_gps_g.reshape(1, D), ln_gps_b.reshape(1, D))


def kernel(x, edge_index, edge_values, W_lin, b_lin, W_agg, Wk, Wq, Wv,
           ln_attn_g, ln_attn_b, W_final, b_final, ln_gps_g, ln_gps_b):
    pad = EPAD - E
    src3 = jnp.pad(edge_index[1], (0, pad)).reshape(NS, NCHUNK, C)
    dst3 = jnp.pad(edge_index[0], (0, pad)).reshape(NS, NCHUNK, C)
    val3 = jnp.pad(edge_values, (0, pad)).reshape(NS, NCHUNK, C)
    spmm = _make_spmm()
    ya, yb = spmm(src3, dst3, val3, x[0], x[1])
    y2a, y2b = spmm(src3, dst3, val3, ya, yb)
    return _epilogue(x, ya, yb, y2a, y2b, W_lin, W_agg, Wk, Wq, Wv, W_final,
                     b_lin, b_final, ln_attn_g, ln_attn_b, ln_gps_g, ln_gps_b)
